# baseline (speedup 1.0000x reference)
import functools

import jax
import jax.numpy as jnp
from jax import lax
from jax.experimental import pallas as pl
from jax.experimental.pallas import tpu as pltpu
from jax.experimental.pallas import tpu_sc as plsc

BATCH = 16384
EMBED = 32
NC = 2
NS = 16
NW = NC * NS
BPW = BATCH // NW
CHUNK = 128
NCHUNK = BPW // CHUNK
L = 16
NGROUP = BPW // L
GATHER_BYTES = 3 * EMBED * BPW * 4


def _sc_body(ut, itm, u_h, i_h, j_h, dummy_h, x_hbm, qu, qi, qj, ru, ri, rj, x_v, sem):
    wid = lax.axis_index("s") * NC + lax.axis_index("c")

    pltpu.sync_copy(u_h.at[wid], qu)
    pltpu.sync_copy(i_h.at[wid], qi)
    pltpu.sync_copy(j_h.at[wid], qj)

    def dma_body(e, carry):
        for k in range(NCHUNK):
            sl = pl.ds(k * CHUNK, CHUNK)
            pltpu.async_copy(ut.at[e].at[qu.at[k]], ru.at[e].at[sl], sem)
            pltpu.async_copy(itm.at[e].at[qi.at[k]], ri.at[e].at[sl], sem)
            pltpu.async_copy(itm.at[e].at[qj.at[k]], rj.at[e].at[sl], sem)
        return carry

    lax.fori_loop(0, EMBED, dma_body, 0)
    pltpu.make_async_copy(dummy_h, ru, sem).wait()
    pltpu.make_async_copy(dummy_h, ri, sem).wait()
    pltpu.make_async_copy(dummy_h, rj, sem).wait()

    def group_body(g, carry):
        sl = pl.ds(g * L, L)
        acc = jnp.zeros((L,), jnp.float32)
        for e in range(EMBED):
            acc = acc + ru[e, sl] * (ri[e, sl] - rj[e, sl])
        x_v[sl] = acc
        return carry

    lax.fori_loop(0, NGROUP, group_body, 0)

    pltpu.sync_copy(x_v, x_hbm.at[pl.ds(wid * BPW, BPW)])


@functools.lru_cache(maxsize=1)
def _make_sc_scores():
    return pl.kernel(
        _sc_body,
        out_type=jax.ShapeDtypeStruct((BATCH,), jnp.float32),
        mesh=plsc.VectorSubcoreMesh(
            core_axis_name="c", subcore_axis_name="s", num_cores=NC, num_subcores=NS
        ),
        compiler_params=pltpu.CompilerParams(
            needs_layout_passes=False, use_tc_tiling_on_sc=False
        ),
        scratch_types=[
            pltpu.VMEM((NCHUNK, CHUNK), jnp.int32),
            pltpu.VMEM((NCHUNK, CHUNK), jnp.int32),
            pltpu.VMEM((NCHUNK, CHUNK), jnp.int32),
            pltpu.VMEM((EMBED, BPW), jnp.float32),
            pltpu.VMEM((EMBED, BPW), jnp.float32),
            pltpu.VMEM((EMBED, BPW), jnp.float32),
            pltpu.VMEM((BPW,), jnp.float32),
            pltpu.SemaphoreType.DMA,
        ],
    )


def _loss_body(x_ref, o_ref):
    y = -x_ref[...]
    sp = jnp.maximum(y, 0.0) + jnp.log1p(jnp.exp(-jnp.abs(y)))
    o_ref[0, 0] = jnp.sum(sp)


def _tc_loss(x):
    out = pl.pallas_call(
        _loss_body,
        out_shape=jax.ShapeDtypeStruct((1, 1), jnp.float32),
        out_specs=pl.BlockSpec(memory_space=pltpu.SMEM),
    )(x.reshape(BATCH // 128, 128))
    return out[0, 0]


@jax.jit
def kernel(u, i, j, user_matrix, item_matrix):
    ut = user_matrix.T
    itm = item_matrix.T
    u3 = u.astype(jnp.int32).reshape(NW, NCHUNK, CHUNK)
    i3 = i.astype(jnp.int32).reshape(NW, NCHUNK, CHUNK)
    j3 = j.astype(jnp.int32).reshape(NW, NCHUNK, CHUNK)
    dummy = jnp.zeros((EMBED, BPW), jnp.float32)
    x = _make_sc_scores()(ut, itm, u3, i3, j3, dummy)
    return _tc_loss(x)


# speedup vs baseline: 1.6804x; 1.6804x over previous
import functools

import jax
import jax.numpy as jnp
from jax import lax
from jax.experimental import pallas as pl
from jax.experimental.pallas import tpu as pltpu
from jax.experimental.pallas import tpu_sc as plsc

BATCH = 16384
EMBED = 32
ROWS = 1000000
PACK = 128 // EMBED
NC = 2
NS = 16
NW = NC * NS
BPW = BATCH // NW
CHUNK = 128
NCHUNK = BPW // CHUNK
L = 16
GPC = CHUNK // L

TBLK = 512
TGRID = ROWS // TBLK


def _transpose_body(x_ref, o_ref):
    o_ref[...] = x_ref[...].T


def _pack_table(t):
    grid = (ROWS + TBLK - 1) // TBLK
    dense = pl.pallas_call(
        _transpose_body,
        grid=(grid,),
        in_specs=[pl.BlockSpec((EMBED, TBLK), lambda g: (0, g))],
        out_specs=pl.BlockSpec((TBLK, EMBED), lambda g: (g, 0)),
        out_shape=jax.ShapeDtypeStruct((ROWS, EMBED), jnp.float32),
    )(t)
    return dense.reshape(ROWS // PACK, 128)


def _sc_body(user2, item2, qu_h, qi_h, qj_h, mu_h, mi_h, mj_h, x_hbm,
             qu, qi, qj, mu, mi, mj, st_u, st_i, st_j, x_v, sem_a, sem_b):
    wid = lax.axis_index("s") * NC + lax.axis_index("c")

    pltpu.sync_copy(qu_h.at[wid], qu)
    pltpu.sync_copy(qi_h.at[wid], qi)
    pltpu.sync_copy(qj_h.at[wid], qj)
    pltpu.sync_copy(mu_h.at[wid], mu)
    pltpu.sync_copy(mi_h.at[wid], mi)
    pltpu.sync_copy(mj_h.at[wid], mj)

    sems = (sem_a, sem_b)

    def fire(k):
        p = k % 2
        return [
            pltpu.async_copy(user2.at[qu.at[k]], st_u.at[p], sems[p]),
            pltpu.async_copy(item2.at[qi.at[k]], st_i.at[p], sems[p]),
            pltpu.async_copy(item2.at[qj.at[k]], st_j.at[p], sems[p]),
        ]

    lane = lax.iota(jnp.int32, L)
    inflight = fire(0)
    for k in range(NCHUNK):
        nxt = fire(k + 1) if k + 1 < NCHUNK else []
        for c in inflight:
            c.wait()
        inflight = nxt
        p = k % 2
        bu, bi, bj = st_u.at[p], st_i.at[p], st_j.at[p]

        def group_body(gl, carry, bu=bu, bi=bi, bj=bj, k=k):
            r = gl * L + lane
            base = k * CHUNK + gl * L
            off_u = mu[pl.ds(base, L)] * EMBED
            off_i = mi[pl.ds(base, L)] * EMBED
            off_j = mj[pl.ds(base, L)] * EMBED
            acc_ui = jnp.zeros((L,), jnp.float32)
            acc_uj = jnp.zeros((L,), jnp.float32)
            for e in range(EMBED):
                uc = plsc.load_gather(bu, [r, off_u + e])
                ic = plsc.load_gather(bi, [r, off_i + e])
                jc = plsc.load_gather(bj, [r, off_j + e])
                acc_ui = acc_ui + uc * ic
                acc_uj = acc_uj + uc * jc
            x_v[pl.ds(base, L)] = acc_ui - acc_uj
            return carry

        lax.fori_loop(0, GPC, group_body, 0)

    pltpu.sync_copy(x_v, x_hbm.at[pl.ds(wid * BPW, BPW)])


@functools.lru_cache(maxsize=1)
def _make_sc_scores():
    return pl.kernel(
        _sc_body,
        out_type=jax.ShapeDtypeStruct((BATCH,), jnp.float32),
        mesh=plsc.VectorSubcoreMesh(
            core_axis_name="c", subcore_axis_name="s", num_cores=NC, num_subcores=NS
        ),
        compiler_params=pltpu.CompilerParams(needs_layout_passes=False),
        scratch_types=[
            pltpu.VMEM((NCHUNK, CHUNK), jnp.int32),
            pltpu.VMEM((NCHUNK, CHUNK), jnp.int32),
            pltpu.VMEM((NCHUNK, CHUNK), jnp.int32),
            pltpu.VMEM((BPW,), jnp.int32),
            pltpu.VMEM((BPW,), jnp.int32),
            pltpu.VMEM((BPW,), jnp.int32),
            pltpu.VMEM((2, CHUNK, 128), jnp.float32),
            pltpu.VMEM((2, CHUNK, 128), jnp.float32),
            pltpu.VMEM((2, CHUNK, 128), jnp.float32),
            pltpu.VMEM((BPW,), jnp.float32),
            pltpu.SemaphoreType.DMA,
            pltpu.SemaphoreType.DMA,
        ],
    )


def _loss_body(x_ref, o_ref):
    y = -x_ref[...]
    sp = jnp.maximum(y, 0.0) + jnp.log1p(jnp.exp(-jnp.abs(y)))
    o_ref[0, 0] = jnp.sum(sp)


def _tc_loss(x):
    out = pl.pallas_call(
        _loss_body,
        out_shape=jax.ShapeDtypeStruct((1, 1), jnp.float32),
        out_specs=pl.BlockSpec(memory_space=pltpu.SMEM),
    )(x.reshape(BATCH // 128, 128))
    return out[0, 0]


@jax.jit
def kernel(u, i, j, user_matrix, item_matrix):
    user2 = _pack_table(user_matrix.T)
    item2 = _pack_table(item_matrix.T)
    u = u.astype(jnp.int32)
    i = i.astype(jnp.int32)
    j = j.astype(jnp.int32)
    qu = (u // PACK).reshape(NW, NCHUNK, CHUNK)
    qi = (i // PACK).reshape(NW, NCHUNK, CHUNK)
    qj = (j // PACK).reshape(NW, NCHUNK, CHUNK)
    mu = (u % PACK).reshape(NW, BPW)
    mi = (i % PACK).reshape(NW, BPW)
    mj = (j % PACK).reshape(NW, BPW)
    x = _make_sc_scores()(user2, item2, qu, qi, qj, mu, mi, mj)
    return _tc_loss(x)


# speedup vs baseline: 5.5719x; 3.3158x over previous
import functools

import jax
import jax.numpy as jnp
from jax import lax
from jax.experimental import pallas as pl
from jax.experimental.pallas import tpu as pltpu
from jax.experimental.pallas import tpu_sc as plsc

BATCH = 16384
EMBED = 32
ROWS = 1000000
PACK = 128 // EMBED
NC = 2
NS = 16
NW = NC * NS
BPW = BATCH // NW
CHUNK = 128
NCHUNK = BPW // CHUNK
L = 16
GPC = CHUNK // L

TBLK = 512
TGRID = ROWS // TBLK


def _sc_body(user2, item2, qu_h, qi_h, qj_h, mu_h, mi_h, mj_h, x_hbm,
             qu, qi, qj, mu, mi, mj, st_u, st_i, st_j, x_v, sem_a, sem_b):
    wid = lax.axis_index("s") * NC + lax.axis_index("c")

    pltpu.sync_copy(qu_h.at[wid], qu)
    pltpu.sync_copy(qi_h.at[wid], qi)
    pltpu.sync_copy(qj_h.at[wid], qj)
    pltpu.sync_copy(mu_h.at[wid], mu)
    pltpu.sync_copy(mi_h.at[wid], mi)
    pltpu.sync_copy(mj_h.at[wid], mj)

    sems = (sem_a, sem_b)

    def fire(k):
        p = k % 2
        return [
            pltpu.async_copy(user2.at[qu.at[k]], st_u.at[p], sems[p]),
            pltpu.async_copy(item2.at[qi.at[k]], st_i.at[p], sems[p]),
            pltpu.async_copy(item2.at[qj.at[k]], st_j.at[p], sems[p]),
        ]

    lane = lax.iota(jnp.int32, L)
    inflight = fire(0)
    for k in range(NCHUNK):
        nxt = fire(k + 1) if k + 1 < NCHUNK else []
        for c in inflight:
            c.wait()
        inflight = nxt
        p = k % 2
        bu, bi, bj = st_u.at[p], st_i.at[p], st_j.at[p]

        def group_body(gl, carry, bu=bu, bi=bi, bj=bj, k=k):
            r = gl * L + lane
            base = k * CHUNK + gl * L
            off_u = mu[pl.ds(base, L)] * EMBED
            off_i = mi[pl.ds(base, L)] * EMBED
            off_j = mj[pl.ds(base, L)] * EMBED
            acc_ui = jnp.zeros((L,), jnp.float32)
            acc_uj = jnp.zeros((L,), jnp.float32)
            for e in range(EMBED):
                uc = plsc.load_gather(bu, [r, off_u + e])
                ic = plsc.load_gather(bi, [r, off_i + e])
                jc = plsc.load_gather(bj, [r, off_j + e])
                acc_ui = acc_ui + uc * ic
                acc_uj = acc_uj + uc * jc
            x_v[pl.ds(base, L)] = acc_ui - acc_uj
            return carry

        lax.fori_loop(0, GPC, group_body, 0)

    pltpu.sync_copy(x_v, x_hbm.at[pl.ds(wid * BPW, BPW)])


@functools.lru_cache(maxsize=1)
def _make_sc_scores():
    return pl.kernel(
        _sc_body,
        out_type=jax.ShapeDtypeStruct((BATCH,), jnp.float32),
        mesh=plsc.VectorSubcoreMesh(
            core_axis_name="c", subcore_axis_name="s", num_cores=NC, num_subcores=NS
        ),
        compiler_params=pltpu.CompilerParams(needs_layout_passes=False),
        scratch_types=[
            pltpu.VMEM((NCHUNK, CHUNK), jnp.int32),
            pltpu.VMEM((NCHUNK, CHUNK), jnp.int32),
            pltpu.VMEM((NCHUNK, CHUNK), jnp.int32),
            pltpu.VMEM((BPW,), jnp.int32),
            pltpu.VMEM((BPW,), jnp.int32),
            pltpu.VMEM((BPW,), jnp.int32),
            pltpu.VMEM((2, CHUNK, 128), jnp.float32),
            pltpu.VMEM((2, CHUNK, 128), jnp.float32),
            pltpu.VMEM((2, CHUNK, 128), jnp.float32),
            pltpu.VMEM((BPW,), jnp.float32),
            pltpu.SemaphoreType.DMA,
            pltpu.SemaphoreType.DMA,
        ],
    )


def _loss_body(x_ref, o_ref):
    y = -x_ref[...]
    sp = jnp.maximum(y, 0.0) + jnp.log1p(jnp.exp(-jnp.abs(y)))
    o_ref[0, 0] = jnp.sum(sp)


def _tc_loss(x):
    out = pl.pallas_call(
        _loss_body,
        out_shape=jax.ShapeDtypeStruct((1, 1), jnp.float32),
        out_specs=pl.BlockSpec(memory_space=pltpu.SMEM),
    )(x.reshape(BATCH // 128, 128))
    return out[0, 0]


@jax.jit
def kernel(u, i, j, user_matrix, item_matrix):
    user2 = user_matrix.reshape(ROWS // PACK, 128)
    item2 = item_matrix.reshape(ROWS // PACK, 128)
    u = u.astype(jnp.int32)
    i = i.astype(jnp.int32)
    j = j.astype(jnp.int32)
    qu = (u // PACK).reshape(NW, NCHUNK, CHUNK)
    qi = (i // PACK).reshape(NW, NCHUNK, CHUNK)
    qj = (j // PACK).reshape(NW, NCHUNK, CHUNK)
    mu = (u % PACK).reshape(NW, BPW)
    mi = (i % PACK).reshape(NW, BPW)
    mj = (j % PACK).reshape(NW, BPW)
    x = _make_sc_scores()(user2, item2, qu, qi, qj, mu, mi, mj)
    return _tc_loss(x)


# speedup vs baseline: 5.7943x; 1.0399x over previous
import functools

import jax
import jax.numpy as jnp
from jax import lax
from jax.experimental import pallas as pl
from jax.experimental.pallas import tpu as pltpu
from jax.experimental.pallas import tpu_sc as plsc

BATCH = 16384
EMBED = 32
ROWS = 1000000
PACK = 128 // EMBED
NC = 2
NS = 16
NW = NC * NS
BPW = BATCH // NW
CHUNK = 128
NCHUNK = BPW // CHUNK
L = 16
GPC = CHUNK // L

TBLK = 512
PGRID = (ROWS + PACK * TBLK - 1) // (PACK * TBLK)
QROWS_P = PGRID * TBLK
NBLK_IN = (ROWS + TBLK - 1) // TBLK


def _relayout_body(x0_ref, x1_ref, x2_ref, x3_ref, o_ref):
    r = lax.broadcasted_iota(jnp.int32, (EMBED, EMBED), 0)
    c = lax.broadcasted_iota(jnp.int32, (EMBED, EMBED), 1)
    ident = (r == c).astype(jnp.float32)
    dn = (((0,), (0,)), ((), ()))
    for m, x_ref in enumerate((x0_ref, x1_ref, x2_ref, x3_ref)):
        o_ref[:, m * EMBED:(m + 1) * EMBED] = jax.lax.dot_general(
            x_ref[...], ident, dn, preferred_element_type=jnp.float32
        )


def _pack_table(t):
    def make_map(m):
        return lambda g: (0, jnp.minimum(PACK * g + m, NBLK_IN - 1))

    bs = [pl.BlockSpec((EMBED, TBLK), make_map(m)) for m in range(PACK)]
    return pl.pallas_call(
        _relayout_body,
        grid=(PGRID,),
        in_specs=bs,
        out_specs=pl.BlockSpec((TBLK, 128), lambda g: (g, 0)),
        out_shape=jax.ShapeDtypeStruct((QROWS_P, 128), jnp.float32),
    )(t, t, t, t)


def _sc_body(user2, item2, qu_h, qi_h, qj_h, mu_h, mi_h, mj_h, x_hbm,
             qu, qi, qj, mu, mi, mj, st_u, st_i, st_j, x_v, sem_a, sem_b):
    wid = lax.axis_index("s") * NC + lax.axis_index("c")

    pltpu.sync_copy(qu_h.at[wid], qu)
    pltpu.sync_copy(qi_h.at[wid], qi)
    pltpu.sync_copy(qj_h.at[wid], qj)
    pltpu.sync_copy(mu_h.at[wid], mu)
    pltpu.sync_copy(mi_h.at[wid], mi)
    pltpu.sync_copy(mj_h.at[wid], mj)

    sems = (sem_a, sem_b)

    def fire(k):
        p = k % 2
        return [
            pltpu.async_copy(user2.at[qu.at[k]], st_u.at[p], sems[p]),
            pltpu.async_copy(item2.at[qi.at[k]], st_i.at[p], sems[p]),
            pltpu.async_copy(item2.at[qj.at[k]], st_j.at[p], sems[p]),
        ]

    lane = lax.iota(jnp.int32, L)
    inflight = fire(0)
    for k in range(NCHUNK):
        nxt = fire(k + 1) if k + 1 < NCHUNK else []
        for c in inflight:
            c.wait()
        inflight = nxt
        p = k % 2
        bu, bi, bj = st_u.at[p], st_i.at[p], st_j.at[p]

        def group_body(gl, carry, bu=bu, bi=bi, bj=bj, k=k):
            r = gl * L + lane
            base = k * CHUNK + gl * L
            off_u = mu[pl.ds(base, L)] * EMBED
            off_i = mi[pl.ds(base, L)] * EMBED
            off_j = mj[pl.ds(base, L)] * EMBED
            acc_ui = jnp.zeros((L,), jnp.float32)
            acc_uj = jnp.zeros((L,), jnp.float32)
            for e in range(EMBED):
                uc = plsc.load_gather(bu, [r, off_u + e])
                ic = plsc.load_gather(bi, [r, off_i + e])
                jc = plsc.load_gather(bj, [r, off_j + e])
                acc_ui = acc_ui + uc * ic
                acc_uj = acc_uj + uc * jc
            x_v[pl.ds(base, L)] = acc_ui - acc_uj
            return carry

        lax.fori_loop(0, GPC, group_body, 0)

    pltpu.sync_copy(x_v, x_hbm.at[pl.ds(wid * BPW, BPW)])


@functools.lru_cache(maxsize=1)
def _make_sc_scores():
    return pl.kernel(
        _sc_body,
        out_type=jax.ShapeDtypeStruct((BATCH,), jnp.float32),
        mesh=plsc.VectorSubcoreMesh(
            core_axis_name="c", subcore_axis_name="s", num_cores=NC, num_subcores=NS
        ),
        compiler_params=pltpu.CompilerParams(needs_layout_passes=False),
        scratch_types=[
            pltpu.VMEM((NCHUNK, CHUNK), jnp.int32),
            pltpu.VMEM((NCHUNK, CHUNK), jnp.int32),
            pltpu.VMEM((NCHUNK, CHUNK), jnp.int32),
            pltpu.VMEM((BPW,), jnp.int32),
            pltpu.VMEM((BPW,), jnp.int32),
            pltpu.VMEM((BPW,), jnp.int32),
            pltpu.VMEM((2, CHUNK, 128), jnp.float32),
            pltpu.VMEM((2, CHUNK, 128), jnp.float32),
            pltpu.VMEM((2, CHUNK, 128), jnp.float32),
            pltpu.VMEM((BPW,), jnp.float32),
            pltpu.SemaphoreType.DMA,
            pltpu.SemaphoreType.DMA,
        ],
    )


def _loss_body(x_ref, o_ref):
    y = -x_ref[...]
    sp = jnp.maximum(y, 0.0) + jnp.log1p(jnp.exp(-jnp.abs(y)))
    o_ref[0, 0] = jnp.sum(sp)


def _tc_loss(x):
    out = pl.pallas_call(
        _loss_body,
        out_shape=jax.ShapeDtypeStruct((1, 1), jnp.float32),
        out_specs=pl.BlockSpec(memory_space=pltpu.SMEM),
    )(x.reshape(BATCH // 128, 128))
    return out[0, 0]


@jax.jit
def kernel(u, i, j, user_matrix, item_matrix):
    user2 = _pack_table(user_matrix.T)
    item2 = _pack_table(item_matrix.T)
    u = u.astype(jnp.int32)
    i = i.astype(jnp.int32)
    j = j.astype(jnp.int32)
    def pack_idx(v):
        return ((v >> 11) << 9) + (v & 511), (v >> 9) & 3

    qu_f, mu_f = pack_idx(u)
    qi_f, mi_f = pack_idx(i)
    qj_f, mj_f = pack_idx(j)
    qu = qu_f.reshape(NW, NCHUNK, CHUNK)
    qi = qi_f.reshape(NW, NCHUNK, CHUNK)
    qj = qj_f.reshape(NW, NCHUNK, CHUNK)
    mu = mu_f.reshape(NW, BPW)
    mi = mi_f.reshape(NW, BPW)
    mj = mj_f.reshape(NW, BPW)
    x = _make_sc_scores()(user2, item2, qu, qi, qj, mu, mi, mj)
    return _tc_loss(x)


# speedup vs baseline: 9.0793x; 1.5669x over previous
import functools

import jax
import jax.numpy as jnp
from jax import lax
from jax.experimental import pallas as pl
from jax.experimental.pallas import tpu as pltpu
from jax.experimental.pallas import tpu_sc as plsc

BATCH = 16384
EMBED = 32
ROWS = 1000000
PACK = 128 // EMBED
NC = 2
NS = 16
NW = NC * NS
BPW = BATCH // NW
CHUNK = 128
NCHUNK = BPW // CHUNK
L = 16
GPC = CHUNK // L

TBLK = 2048
PGRID = (ROWS + PACK * TBLK - 1) // (PACK * TBLK)
QROWS_P = PGRID * TBLK
NBLK_IN = (ROWS + TBLK - 1) // TBLK


def _relayout_body(x0_ref, x1_ref, x2_ref, x3_ref, o_ref):
    r = lax.broadcasted_iota(jnp.int32, (EMBED, EMBED), 0)
    c = lax.broadcasted_iota(jnp.int32, (EMBED, EMBED), 1)
    ident = (r == c).astype(jnp.float32)
    dn = (((0,), (0,)), ((), ()))
    for m, x_ref in enumerate((x0_ref, x1_ref, x2_ref, x3_ref)):
        o_ref[:, m * EMBED:(m + 1) * EMBED] = jax.lax.dot_general(
            x_ref[...], ident, dn, preferred_element_type=jnp.float32
        )


def _pack_table(t):
    def make_map(m):
        return lambda g: (0, jnp.minimum(PACK * g + m, NBLK_IN - 1))

    bs = [pl.BlockSpec((EMBED, TBLK), make_map(m)) for m in range(PACK)]
    return pl.pallas_call(
        _relayout_body,
        grid=(PGRID,),
        in_specs=bs,
        out_specs=pl.BlockSpec((TBLK, 128), lambda g: (g, 0)),
        out_shape=jax.ShapeDtypeStruct((QROWS_P, 128), jnp.float32),
    )(t, t, t, t)


def _sc_body(user2, item2, qu_h, qi_h, qj_h, mu_h, mi_h, mj_h, x_hbm,
             qu, qi, qj, mu, mi, mj, st_u, st_i, st_j, x_v, sem_a, sem_b):
    wid = lax.axis_index("s") * NC + lax.axis_index("c")

    pltpu.sync_copy(qu_h.at[wid], qu)
    pltpu.sync_copy(qi_h.at[wid], qi)
    pltpu.sync_copy(qj_h.at[wid], qj)
    pltpu.sync_copy(mu_h.at[wid], mu)
    pltpu.sync_copy(mi_h.at[wid], mi)
    pltpu.sync_copy(mj_h.at[wid], mj)

    sems = (sem_a, sem_b)

    def fire(k):
        p = k % 2
        return [
            pltpu.async_copy(user2.at[qu.at[k]], st_u.at[p], sems[p]),
            pltpu.async_copy(item2.at[qi.at[k]], st_i.at[p], sems[p]),
            pltpu.async_copy(item2.at[qj.at[k]], st_j.at[p], sems[p]),
        ]

    lane = lax.iota(jnp.int32, L)
    inflight = fire(0)
    for k in range(NCHUNK):
        nxt = fire(k + 1) if k + 1 < NCHUNK else []
        for c in inflight:
            c.wait()
        inflight = nxt
        p = k % 2
        bu, bi, bj = st_u.at[p], st_i.at[p], st_j.at[p]

        def group_body(gl, carry, bu=bu, bi=bi, bj=bj, k=k):
            r = gl * L + lane
            base = k * CHUNK + gl * L
            off_u = mu[pl.ds(base, L)] * EMBED
            off_i = mi[pl.ds(base, L)] * EMBED
            off_j = mj[pl.ds(base, L)] * EMBED
            acc_ui = jnp.zeros((L,), jnp.float32)
            acc_uj = jnp.zeros((L,), jnp.float32)
            for e in range(EMBED):
                uc = plsc.load_gather(bu, [r, off_u + e])
                ic = plsc.load_gather(bi, [r, off_i + e])
                jc = plsc.load_gather(bj, [r, off_j + e])
                acc_ui = acc_ui + uc * ic
                acc_uj = acc_uj + uc * jc
            x_v[pl.ds(base, L)] = acc_ui - acc_uj
            return carry

        lax.fori_loop(0, GPC, group_body, 0)

    pltpu.sync_copy(x_v, x_hbm.at[pl.ds(wid * BPW, BPW)])


@functools.lru_cache(maxsize=1)
def _make_sc_scores():
    return pl.kernel(
        _sc_body,
        out_type=jax.ShapeDtypeStruct((BATCH,), jnp.float32),
        mesh=plsc.VectorSubcoreMesh(
            core_axis_name="c", subcore_axis_name="s", num_cores=NC, num_subcores=NS
        ),
        compiler_params=pltpu.CompilerParams(needs_layout_passes=False),
        scratch_types=[
            pltpu.VMEM((NCHUNK, CHUNK), jnp.int32),
            pltpu.VMEM((NCHUNK, CHUNK), jnp.int32),
            pltpu.VMEM((NCHUNK, CHUNK), jnp.int32),
            pltpu.VMEM((BPW,), jnp.int32),
            pltpu.VMEM((BPW,), jnp.int32),
            pltpu.VMEM((BPW,), jnp.int32),
            pltpu.VMEM((2, CHUNK, 128), jnp.float32),
            pltpu.VMEM((2, CHUNK, 128), jnp.float32),
            pltpu.VMEM((2, CHUNK, 128), jnp.float32),
            pltpu.VMEM((BPW,), jnp.float32),
            pltpu.SemaphoreType.DMA,
            pltpu.SemaphoreType.DMA,
        ],
    )


def _loss_body(x_ref, o_ref):
    y = -x_ref[...]
    sp = jnp.maximum(y, 0.0) + jnp.log1p(jnp.exp(-jnp.abs(y)))
    o_ref[0, 0] = jnp.sum(sp)


def _tc_loss(x):
    out = pl.pallas_call(
        _loss_body,
        out_shape=jax.ShapeDtypeStruct((1, 1), jnp.float32),
        out_specs=pl.BlockSpec(memory_space=pltpu.SMEM),
    )(x.reshape(BATCH // 128, 128))
    return out[0, 0]


@jax.jit
def kernel(u, i, j, user_matrix, item_matrix):
    user2 = _pack_table(user_matrix.T)
    item2 = _pack_table(item_matrix.T)
    u = u.astype(jnp.int32)
    i = i.astype(jnp.int32)
    j = j.astype(jnp.int32)
    def pack_idx(v):
        return ((v >> 13) << 11) + (v & 2047), (v >> 11) & 3

    qu_f, mu_f = pack_idx(u)
    qi_f, mi_f = pack_idx(i)
    qj_f, mj_f = pack_idx(j)
    qu = qu_f.reshape(NW, NCHUNK, CHUNK)
    qi = qi_f.reshape(NW, NCHUNK, CHUNK)
    qj = qj_f.reshape(NW, NCHUNK, CHUNK)
    mu = mu_f.reshape(NW, BPW)
    mi = mi_f.reshape(NW, BPW)
    mj = mj_f.reshape(NW, BPW)
    x = _make_sc_scores()(user2, item2, qu, qi, qj, mu, mi, mj)
    return _tc_loss(x)


# speedup vs baseline: 9.4375x; 1.0395x over previous
import functools

import jax
import jax.numpy as jnp
from jax import lax
from jax.experimental import pallas as pl
from jax.experimental.pallas import tpu as pltpu
from jax.experimental.pallas import tpu_sc as plsc

BATCH = 16384
EMBED = 32
ROWS = 1000000
PACK = 128 // EMBED
NC = 2
NS = 16
NW = NC * NS
BPW = BATCH // NW
CHUNK = 128
NCHUNK = BPW // CHUNK
L = 16
GPC = CHUNK // L

TBLK = 8192
PGRID = (ROWS + PACK * TBLK - 1) // (PACK * TBLK)
QROWS_P = PGRID * TBLK
NBLK_IN = (ROWS + TBLK - 1) // TBLK


def _relayout_body(x0_ref, x1_ref, x2_ref, x3_ref, o_ref):
    r = lax.broadcasted_iota(jnp.int32, (EMBED, EMBED), 0)
    c = lax.broadcasted_iota(jnp.int32, (EMBED, EMBED), 1)
    ident = (r == c).astype(jnp.float32)
    dn = (((0,), (0,)), ((), ()))
    for m, x_ref in enumerate((x0_ref, x1_ref, x2_ref, x3_ref)):
        o_ref[:, m * EMBED:(m + 1) * EMBED] = jax.lax.dot_general(
            x_ref[...], ident, dn, preferred_element_type=jnp.float32
        )


def _pack_table(t):
    def make_map(m):
        return lambda g: (0, jnp.minimum(PACK * g + m, NBLK_IN - 1))

    bs = [pl.BlockSpec((EMBED, TBLK), make_map(m)) for m in range(PACK)]
    return pl.pallas_call(
        _relayout_body,
        grid=(PGRID,),
        in_specs=bs,
        out_specs=pl.BlockSpec((TBLK, 128), lambda g: (g, 0)),
        out_shape=jax.ShapeDtypeStruct((QROWS_P, 128), jnp.float32),
    )(t, t, t, t)


def _sc_body(user2, item2, qu_h, qi_h, qj_h, mu_h, mi_h, mj_h, x_hbm,
             qu, qi, qj, mu, mi, mj, st_u, st_i, st_j, x_v, sem_a, sem_b):
    wid = lax.axis_index("s") * NC + lax.axis_index("c")

    pltpu.sync_copy(qu_h.at[wid], qu)
    pltpu.sync_copy(qi_h.at[wid], qi)
    pltpu.sync_copy(qj_h.at[wid], qj)
    pltpu.sync_copy(mu_h.at[wid], mu)
    pltpu.sync_copy(mi_h.at[wid], mi)
    pltpu.sync_copy(mj_h.at[wid], mj)

    sems = (sem_a, sem_b)

    def fire(k):
        p = k % 2
        return [
            pltpu.async_copy(user2.at[qu.at[k]], st_u.at[p], sems[p]),
            pltpu.async_copy(item2.at[qi.at[k]], st_i.at[p], sems[p]),
            pltpu.async_copy(item2.at[qj.at[k]], st_j.at[p], sems[p]),
        ]

    lane = lax.iota(jnp.int32, L)
    inflight = fire(0)
    for k in range(NCHUNK):
        nxt = fire(k + 1) if k + 1 < NCHUNK else []
        for c in inflight:
            c.wait()
        inflight = nxt
        p = k % 2
        bu, bi, bj = st_u.at[p], st_i.at[p], st_j.at[p]

        def group_body(gl, carry, bu=bu, bi=bi, bj=bj, k=k):
            r = gl * L + lane
            base = k * CHUNK + gl * L
            off_u = mu[pl.ds(base, L)] * EMBED
            off_i = mi[pl.ds(base, L)] * EMBED
            off_j = mj[pl.ds(base, L)] * EMBED
            acc_ui = jnp.zeros((L,), jnp.float32)
            acc_uj = jnp.zeros((L,), jnp.float32)
            for e in range(EMBED):
                uc = plsc.load_gather(bu, [r, off_u + e])
                ic = plsc.load_gather(bi, [r, off_i + e])
                jc = plsc.load_gather(bj, [r, off_j + e])
                acc_ui = acc_ui + uc * ic
                acc_uj = acc_uj + uc * jc
            x_v[pl.ds(base, L)] = acc_ui - acc_uj
            return carry

        lax.fori_loop(0, GPC, group_body, 0)

    pltpu.sync_copy(x_v, x_hbm.at[pl.ds(wid * BPW, BPW)])


@functools.lru_cache(maxsize=1)
def _make_sc_scores():
    return pl.kernel(
        _sc_body,
        out_type=jax.ShapeDtypeStruct((BATCH,), jnp.float32),
        mesh=plsc.VectorSubcoreMesh(
            core_axis_name="c", subcore_axis_name="s", num_cores=NC, num_subcores=NS
        ),
        compiler_params=pltpu.CompilerParams(needs_layout_passes=False),
        scratch_types=[
            pltpu.VMEM((NCHUNK, CHUNK), jnp.int32),
            pltpu.VMEM((NCHUNK, CHUNK), jnp.int32),
            pltpu.VMEM((NCHUNK, CHUNK), jnp.int32),
            pltpu.VMEM((BPW,), jnp.int32),
            pltpu.VMEM((BPW,), jnp.int32),
            pltpu.VMEM((BPW,), jnp.int32),
            pltpu.VMEM((2, CHUNK, 128), jnp.float32),
            pltpu.VMEM((2, CHUNK, 128), jnp.float32),
            pltpu.VMEM((2, CHUNK, 128), jnp.float32),
            pltpu.VMEM((BPW,), jnp.float32),
            pltpu.SemaphoreType.DMA,
            pltpu.SemaphoreType.DMA,
        ],
    )


def _loss_body(x_ref, o_ref):
    y = -x_ref[...]
    sp = jnp.maximum(y, 0.0) + jnp.log1p(jnp.exp(-jnp.abs(y)))
    o_ref[0, 0] = jnp.sum(sp)


def _tc_loss(x):
    out = pl.pallas_call(
        _loss_body,
        out_shape=jax.ShapeDtypeStruct((1, 1), jnp.float32),
        out_specs=pl.BlockSpec(memory_space=pltpu.SMEM),
    )(x.reshape(BATCH // 128, 128))
    return out[0, 0]


@jax.jit
def kernel(u, i, j, user_matrix, item_matrix):
    user2 = _pack_table(user_matrix.T)
    item2 = _pack_table(item_matrix.T)
    u = u.astype(jnp.int32)
    i = i.astype(jnp.int32)
    j = j.astype(jnp.int32)
    def pack_idx(v):
        return ((v >> 15) << 13) + (v & 8191), (v >> 13) & 3

    qu_f, mu_f = pack_idx(u)
    qi_f, mi_f = pack_idx(i)
    qj_f, mj_f = pack_idx(j)
    qu = qu_f.reshape(NW, NCHUNK, CHUNK)
    qi = qi_f.reshape(NW, NCHUNK, CHUNK)
    qj = qj_f.reshape(NW, NCHUNK, CHUNK)
    mu = mu_f.reshape(NW, BPW)
    mi = mi_f.reshape(NW, BPW)
    mj = mj_f.reshape(NW, BPW)
    x = _make_sc_scores()(user2, item2, qu, qi, qj, mu, mi, mj)
    return _tc_loss(x)


# speedup vs baseline: 9.4463x; 1.0009x over previous
import functools

import jax
import jax.numpy as jnp
from jax import lax
from jax.experimental import pallas as pl
from jax.experimental.pallas import tpu as pltpu
from jax.experimental.pallas import tpu_sc as plsc

BATCH = 16384
EMBED = 32
ROWS = 1000000
PACK = 128 // EMBED
NC = 2
NS = 16
NW = NC * NS
BPW = BATCH // NW
CHUNK = 128
NCHUNK = BPW // CHUNK
L = 16
GPC = CHUNK // L

TBLK = 8192
PGRID = (ROWS + PACK * TBLK - 1) // (PACK * TBLK)
QROWS_P = PGRID * TBLK
NBLK_IN = (ROWS + TBLK - 1) // TBLK


def _relayout_body(x0_ref, x1_ref, x2_ref, x3_ref, o_ref):
    r = lax.broadcasted_iota(jnp.int32, (EMBED, EMBED), 0)
    c = lax.broadcasted_iota(jnp.int32, (EMBED, EMBED), 1)
    ident = (r == c).astype(jnp.float32)
    dn = (((0,), (0,)), ((), ()))
    for m, x_ref in enumerate((x0_ref, x1_ref, x2_ref, x3_ref)):
        o_ref[:, m * EMBED:(m + 1) * EMBED] = jax.lax.dot_general(
            x_ref[...], ident, dn, preferred_element_type=jnp.float32
        )


def _pack_table(t):
    def make_map(m):
        return lambda g: (0, jnp.minimum(PACK * g + m, NBLK_IN - 1))

    bs = [pl.BlockSpec((EMBED, TBLK), make_map(m)) for m in range(PACK)]
    return pl.pallas_call(
        _relayout_body,
        grid=(PGRID,),
        in_specs=bs,
        out_specs=pl.BlockSpec((TBLK, 128), lambda g: (g, 0)),
        out_shape=jax.ShapeDtypeStruct((QROWS_P, 128), jnp.float32),
        compiler_params=pltpu.CompilerParams(fuse_transposed_lhs_in_matmul=True),
    )(t, t, t, t)


def _sc_body(user2, item2, qu_h, qi_h, qj_h, mu_h, mi_h, mj_h, x_hbm,
             qu, qi, qj, mu, mi, mj, st_u, st_i, st_j, x_v, sem_a, sem_b):
    wid = lax.axis_index("s") * NC + lax.axis_index("c")

    pltpu.sync_copy(qu_h.at[wid], qu)
    pltpu.sync_copy(qi_h.at[wid], qi)
    pltpu.sync_copy(qj_h.at[wid], qj)
    pltpu.sync_copy(mu_h.at[wid], mu)
    pltpu.sync_copy(mi_h.at[wid], mi)
    pltpu.sync_copy(mj_h.at[wid], mj)

    sems = (sem_a, sem_b)

    def fire(k):
        p = k % 2
        return [
            pltpu.async_copy(user2.at[qu.at[k]], st_u.at[p], sems[p]),
            pltpu.async_copy(item2.at[qi.at[k]], st_i.at[p], sems[p]),
            pltpu.async_copy(item2.at[qj.at[k]], st_j.at[p], sems[p]),
        ]

    lane = lax.iota(jnp.int32, L)
    inflight = fire(0)
    for k in range(NCHUNK):
        nxt = fire(k + 1) if k + 1 < NCHUNK else []
        for c in inflight:
            c.wait()
        inflight = nxt
        p = k % 2
        bu, bi, bj = st_u.at[p], st_i.at[p], st_j.at[p]

        def group_body(gl, carry, bu=bu, bi=bi, bj=bj, k=k):
            r = gl * L + lane
            base = k * CHUNK + gl * L
            off_u = mu[pl.ds(base, L)] * EMBED
            off_i = mi[pl.ds(base, L)] * EMBED
            off_j = mj[pl.ds(base, L)] * EMBED
            acc_ui = jnp.zeros((L,), jnp.float32)
            acc_uj = jnp.zeros((L,), jnp.float32)
            for e in range(EMBED):
                uc = plsc.load_gather(bu, [r, off_u + e])
                ic = plsc.load_gather(bi, [r, off_i + e])
                jc = plsc.load_gather(bj, [r, off_j + e])
                acc_ui = acc_ui + uc * ic
                acc_uj = acc_uj + uc * jc
            x_v[pl.ds(base, L)] = acc_ui - acc_uj
            return carry

        lax.fori_loop(0, GPC, group_body, 0)

    pltpu.sync_copy(x_v, x_hbm.at[pl.ds(wid * BPW, BPW)])


@functools.lru_cache(maxsize=1)
def _make_sc_scores():
    return pl.kernel(
        _sc_body,
        out_type=jax.ShapeDtypeStruct((BATCH,), jnp.float32),
        mesh=plsc.VectorSubcoreMesh(
            core_axis_name="c", subcore_axis_name="s", num_cores=NC, num_subcores=NS
        ),
        compiler_params=pltpu.CompilerParams(needs_layout_passes=False),
        scratch_types=[
            pltpu.VMEM((NCHUNK, CHUNK), jnp.int32),
            pltpu.VMEM((NCHUNK, CHUNK), jnp.int32),
            pltpu.VMEM((NCHUNK, CHUNK), jnp.int32),
            pltpu.VMEM((BPW,), jnp.int32),
            pltpu.VMEM((BPW,), jnp.int32),
            pltpu.VMEM((BPW,), jnp.int32),
            pltpu.VMEM((2, CHUNK, 128), jnp.float32),
            pltpu.VMEM((2, CHUNK, 128), jnp.float32),
            pltpu.VMEM((2, CHUNK, 128), jnp.float32),
            pltpu.VMEM((BPW,), jnp.float32),
            pltpu.SemaphoreType.DMA,
            pltpu.SemaphoreType.DMA,
        ],
    )


def _loss_body(x_ref, o_ref):
    y = -x_ref[...]
    sp = jnp.maximum(y, 0.0) + jnp.log1p(jnp.exp(-jnp.abs(y)))
    o_ref[0, 0] = jnp.sum(sp)


def _tc_loss(x):
    out = pl.pallas_call(
        _loss_body,
        out_shape=jax.ShapeDtypeStruct((1, 1), jnp.float32),
        out_specs=pl.BlockSpec(memory_space=pltpu.SMEM),
    )(x.reshape(BATCH // 128, 128))
    return out[0, 0]


@jax.jit
def kernel(u, i, j, user_matrix, item_matrix):
    user2 = _pack_table(user_matrix.T)
    item2 = _pack_table(item_matrix.T)
    u = u.astype(jnp.int32)
    i = i.astype(jnp.int32)
    j = j.astype(jnp.int32)
    def pack_idx(v):
        return ((v >> 15) << 13) + (v & 8191), (v >> 13) & 3

    qu_f, mu_f = pack_idx(u)
    qi_f, mi_f = pack_idx(i)
    qj_f, mj_f = pack_idx(j)
    qu = qu_f.reshape(NW, NCHUNK, CHUNK)
    qi = qi_f.reshape(NW, NCHUNK, CHUNK)
    qj = qj_f.reshape(NW, NCHUNK, CHUNK)
    mu = mu_f.reshape(NW, BPW)
    mi = mi_f.reshape(NW, BPW)
    mj = mj_f.reshape(NW, BPW)
    x = _make_sc_scores()(user2, item2, qu, qi, qj, mu, mi, mj)
    return _tc_loss(x)
